# Initial kernel scaffold; baseline (speedup 1.0000x reference)
import jax
import jax.numpy as jnp
from jax.experimental import pallas as pl


def kernel(x, edge_index, W1, b1, W2, b2, W3, b3, Wd, bd):
    raise NotImplementedError("write your pallas kernel here")


# speedup vs baseline: 12.9898x; 12.9898x over previous
import functools

import jax
import jax.numpy as jnp
from jax import lax
from jax.experimental import pallas as pl
from jax.experimental.pallas import tpu as pltpu
from jax.experimental.pallas import tpu_sc as plsc

NC = 2
NS = 16
NW = NC * NS
LANES = 16

CH = 128
NBR = 1024
NBN = 2048


def _sc_mesh():
    return plsc.VectorSubcoreMesh(
        core_axis_name="c", subcore_axis_name="s",
        num_cores=NC, num_subcores=NS)


def kernel(x, edge_index, W1, b1, W2, b2, W3, b3, Wd, bd):
    N, D = x.shape
    E = edge_index.shape[1]
    C = Wd.shape[1]

    N_pad = ((N + LANES + NBR - 1) // NBR) * NBR
    rows_per_tile = N_pad // NS
    K = (E + NW * CH - 1) // (NW * CH)
    E_pad = K * CH * NW

    src = edge_index[0].astype(jnp.int32)
    dst = edge_index[1].astype(jnp.int32)
    pad_ids = N + (jnp.arange(E_pad - E, dtype=jnp.int32) % LANES)
    src3 = jnp.concatenate([src, pad_ids]).reshape(NW, K, CH)
    dst3 = jnp.concatenate([dst, pad_ids]).reshape(NW, K, CH)
    x_pad = jnp.pad(x, ((0, N_pad - N), (0, 0)))
    zrows = jnp.zeros((rows_per_tile, D), jnp.float32)
    b1r, b2r, b3r = b1.reshape(1, D), b2.reshape(1, D), b3.reshape(1, D)
    bdr = bd.reshape(1, C)

    mesh = _sc_mesh()

    @functools.partial(
        pl.kernel,
        out_type=jax.ShapeDtypeStruct((2, NW, N_pad), jnp.float32),
        mesh=mesh,
        scratch_types=[
            pltpu.VMEM((K, CH), jnp.int32),
            pltpu.VMEM((K, CH), jnp.int32),
            pltpu.VMEM((N_pad,), jnp.float32),
            pltpu.VMEM((N_pad,), jnp.float32),
        ],
        compiler_params=pltpu.CompilerParams(needs_layout_passes=False),
    )
    def deg_kernel(src_hbm, dst_hbm, out_hbm, sidx, didx, hs, hd):
        c = lax.axis_index("c")
        s = lax.axis_index("s")
        wid = c * NS + s
        zeros16 = jnp.zeros((LANES,), jnp.float32)

        def zbody(i, _):
            hs[pl.ds(i * LANES, LANES)] = zeros16
            hd[pl.ds(i * LANES, LANES)] = zeros16
            return 0
        lax.fori_loop(0, N_pad // LANES, zbody, 0)

        pltpu.sync_copy(src_hbm.at[wid], sidx)
        pltpu.sync_copy(dst_hbm.at[wid], didx)
        ones16 = jnp.ones((LANES,), jnp.float32)

        def ebody(j, _):
            for k in range(CH // LANES):
                si = sidx[j, pl.ds(k * LANES, LANES)]
                di = didx[j, pl.ds(k * LANES, LANES)]
                plsc.addupdate_scatter(hs, [si], ones16)
                plsc.addupdate_scatter(hd, [di], ones16)
            return 0
        lax.fori_loop(0, K, ebody, 0)

        pltpu.sync_copy(hs, out_hbm.at[0, wid])
        pltpu.sync_copy(hd, out_hbm.at[1, wid])

    degpart = deg_kernel(src3, dst3)

    def norm_body(dp_ref, out_ref):
        s = jnp.sum(dp_ref[...], axis=1)
        out_ref[...] = lax.rsqrt(jnp.maximum(s, 1.0))

    norms_t = pl.pallas_call(
        norm_body,
        grid=(N_pad // NBN,),
        in_specs=[pl.BlockSpec((2, NW, NBN), lambda i: (0, 0, i))],
        out_specs=pl.BlockSpec((2, NBN), lambda i: (0, i)),
        out_shape=jax.ShapeDtypeStruct((2, N_pad), jnp.float32),
    )(degpart)
    norms = norms_t.T

    @functools.partial(
        pl.kernel,
        out_type=jax.ShapeDtypeStruct((NC, N_pad, D), jnp.float32),
        mesh=mesh,
        scratch_types=[
            pltpu.VMEM((K, CH), jnp.int32),
            pltpu.VMEM((K, CH), jnp.int32),
            pltpu.VMEM((CH, D), jnp.float32),
            pltpu.VMEM_SHARED((N_pad, D), jnp.float32),
            pltpu.SemaphoreType.DMA,
        ],
        compiler_params=pltpu.CompilerParams(needs_layout_passes=False),
    )
    def agg_kernel(h_hbm, src_hbm, dst_hbm, z_hbm, out_hbm,
                   sidx, didx, rows, aggm, sem):
        c = lax.axis_index("c")
        s = lax.axis_index("s")
        wid = c * NS + s
        base = s * rows_per_tile
        pltpu.sync_copy(z_hbm, aggm.at[pl.ds(base, rows_per_tile)])
        pltpu.sync_copy(src_hbm.at[wid], sidx)
        pltpu.sync_copy(dst_hbm.at[wid], didx)
        plsc.subcore_barrier()

        def ebody(j, _):
            pltpu.async_copy(h_hbm.at[sidx.at[j]], rows, sem).wait()
            pltpu.sync_copy(rows, aggm.at[didx.at[j]], add=True)
            return 0
        lax.fori_loop(0, K, ebody, 0)

        plsc.subcore_barrier()
        pltpu.sync_copy(aggm.at[pl.ds(base, rows_per_tile)],
                        out_hbm.at[c, pl.ds(base, rows_per_tile)])

    def mm_scale_body(x_ref, w_ref, nrm_ref, out_ref):
        ns = nrm_ref[...][:, 0:1]
        out_ref[...] = jnp.dot(x_ref[...], w_ref[...],
                               preferred_element_type=jnp.float32) * ns

    def mm_scale(h, W):
        return pl.pallas_call(
            mm_scale_body,
            grid=(N_pad // NBR,),
            in_specs=[
                pl.BlockSpec((NBR, D), lambda i: (i, 0)),
                pl.BlockSpec((D, D), lambda i: (0, 0)),
                pl.BlockSpec((NBR, 2), lambda i: (i, 0)),
            ],
            out_specs=pl.BlockSpec((NBR, D), lambda i: (i, 0)),
            out_shape=jax.ShapeDtypeStruct((N_pad, D), jnp.float32),
        )(h, W, norms)

    def epi_mm_body(ap_ref, nrm_ref, b_ref, w_ref, out_ref):
        i = pl.program_id(0)
        agg = ap_ref[0] + ap_ref[1]
        z = agg * nrm_ref[...][:, 1:2] + b_ref[...]
        h = jnp.where(z >= 0.0, z, 0.2 * z)
        row = i * NBR + lax.broadcasted_iota(jnp.int32, (NBR, 1), 0)
        h = jnp.where(row < N, h, 0.0)
        out_ref[...] = jnp.dot(h, w_ref[...],
                               preferred_element_type=jnp.float32) \
            * nrm_ref[...][:, 0:1]

    def epi_mm(ap, b_row, W):
        return pl.pallas_call(
            epi_mm_body,
            grid=(N_pad // NBR,),
            in_specs=[
                pl.BlockSpec((NC, NBR, D), lambda i: (0, i, 0)),
                pl.BlockSpec((NBR, 2), lambda i: (i, 0)),
                pl.BlockSpec((1, D), lambda i: (0, 0)),
                pl.BlockSpec((D, D), lambda i: (0, 0)),
            ],
            out_specs=pl.BlockSpec((NBR, D), lambda i: (i, 0)),
            out_shape=jax.ShapeDtypeStruct((N_pad, D), jnp.float32),
        )(ap, norms, b_row, W)

    def final_body(ap_ref, nrm_ref, b_ref, wd_ref, bd_ref, out_ref, acc_ref):
        i = pl.program_id(0)
        agg = ap_ref[0] + ap_ref[1]
        z = agg * nrm_ref[...][:, 1:2] + b_ref[...]
        h = jnp.where(z >= 0.0, z, 0.2 * z)
        row = i * NBR + lax.broadcasted_iota(jnp.int32, (NBR, 1), 0)
        h = jnp.where(row < N, h, 0.0)

        @pl.when(i == 0)
        def _():
            acc_ref[...] = jnp.zeros_like(acc_ref)
        acc_ref[...] += jnp.sum(h, axis=0, keepdims=True)

        @pl.when(i == N_pad // NBR - 1)
        def _():
            out_ref[...] = jnp.dot(acc_ref[...] * (1.0 / N), wd_ref[...],
                                   preferred_element_type=jnp.float32) \
                + bd_ref[...]

    def final(ap):
        return pl.pallas_call(
            final_body,
            grid=(N_pad // NBR,),
            in_specs=[
                pl.BlockSpec((NC, NBR, D), lambda i: (0, i, 0)),
                pl.BlockSpec((NBR, 2), lambda i: (i, 0)),
                pl.BlockSpec((1, D), lambda i: (0, 0)),
                pl.BlockSpec((D, C), lambda i: (0, 0)),
                pl.BlockSpec((1, C), lambda i: (0, 0)),
            ],
            out_specs=pl.BlockSpec((1, C), lambda i: (0, 0)),
            out_shape=jax.ShapeDtypeStruct((1, C), jnp.float32),
            scratch_shapes=[pltpu.VMEM((1, D), jnp.float32)],
        )(ap, norms, b3r, Wd, bdr)

    ht = mm_scale(x_pad, W1)
    ap = agg_kernel(ht, src3, dst3, zrows)
    ht = epi_mm(ap, b1r, W2)
    ap = agg_kernel(ht, src3, dst3, zrows)
    ht = epi_mm(ap, b2r, W3)
    ap = agg_kernel(ht, src3, dst3, zrows)
    return final(ap)
